# baseline (speedup 1.0000x reference)
import jax
import jax.numpy as jnp
from jax.experimental import pallas as pl
from jax.experimental.pallas import tpu as pltpu

B = 16384
D = 128
CD = 64
K = 1024
BT = 4096
NB = B // BT
N1 = float(B * CD)
N2 = float(B * D)


def _leaky(x):
    return jnp.where(x >= 0, x, 0.1 * x)


def _split_hi_lo(x):
    hi = x.astype(jnp.bfloat16)
    lo = (x - hi.astype(jnp.float32)).astype(jnp.bfloat16)
    return hi, lo


def _make_table(emb, w_t, bias, gamma, beta):
    p = jax.lax.dot_general(
        emb, w_t, (((1,), (0,)), ((), ())), preferred_element_type=jnp.float32
    ) + bias
    m = jnp.mean(p, axis=1, keepdims=True)
    v = jnp.mean((p - m) ** 2, axis=1, keepdims=True)
    pt = (p - m) / jnp.sqrt(v + 1e-5) * gamma + beta
    e_hi, e_lo = _split_hi_lo(emb)
    pt_hi, pt_lo = _split_hi_lo(pt)
    return jnp.concatenate([e_hi, e_lo, pt_hi, pt_lo], axis=1)


def _vq_gather(x_aug, rhs, tbl):
    dist = jax.lax.dot_general(
        x_aug, rhs, (((1,), (0,)), ((), ())), preferred_element_type=jnp.float32
    )
    m = jnp.min(dist, axis=1, keepdims=True)
    onehot = (dist == m).astype(jnp.bfloat16)
    g = jax.lax.dot_general(
        onehot, tbl, (((1,), (0,)), ((), ())), preferred_element_type=jnp.float32
    )
    q = g[:, 0:CD] + g[:, CD:2 * CD]
    proj = _leaky(g[:, 2 * CD:3 * CD] + g[:, 3 * CD:4 * CD])
    return q, proj, m


def _kernel(
    z_ref,
    cemb_ref,
    cembt_ref,
    femb_ref,
    fembt_ref,
    c2f_w_ref,
    c2f_b_ref,
    c2f_g_ref,
    c2f_be_ref,
    f2c_w_ref,
    f2c_b_ref,
    f2c_g_ref,
    f2c_be_ref,
    gates_ref,
    emac_ref,
    emaf_ref,
    zh_ref,
    scal_ref,
    acc_ref,
    tblc_ref,
    tblf_ref,
    cm2_ref,
    fm2_ref,
    xaug_ref,
):
    i = pl.program_id(0)

    @pl.when(i == 0)
    def _init():
        acc_ref[:, :] = jnp.zeros((8, 128), jnp.float32)
        tblc_ref[:, :] = _make_table(
            cemb_ref[:, :], c2f_w_ref[:, :], c2f_b_ref[0:1, :],
            c2f_g_ref[0:1, :], c2f_be_ref[0:1, :],
        )
        tblf_ref[:, :] = _make_table(
            femb_ref[:, :], f2c_w_ref[:, :], f2c_b_ref[0:1, :],
            f2c_g_ref[0:1, :], f2c_be_ref[0:1, :],
        )
        ct = cembt_ref[:, :]
        ft = fembt_ref[:, :]
        cm2_ref[0:CD, :] = -2.0 * ct
        cm2_ref[CD:CD + 1, :] = jnp.sum(ct * ct, axis=0, keepdims=True)
        cm2_ref[CD + 1:, :] = jnp.zeros((7, K), jnp.float32)
        fm2_ref[0:CD, :] = -2.0 * ft
        fm2_ref[CD:CD + 1, :] = jnp.sum(ft * ft, axis=0, keepdims=True)
        fm2_ref[CD + 1:, :] = jnp.zeros((7, K), jnp.float32)
        xaug_ref[:, CD:] = jnp.ones((BT, 8), jnp.float32)

    gate_c = jax.nn.sigmoid(gates_ref[0:1, 0:1])
    gate_f = jax.nn.sigmoid(gates_ref[0:1, 1:2])

    for h in range(1):
        r0, r1 = 0, BT
        zc = z_ref[r0:r1, :CD]
        zf = z_ref[r0:r1, CD:]

        xaug_ref[:, 0:CD] = zc
        zcq, ci, mc = _vq_gather(
            xaug_ref[:, 0:CD + 1], cm2_ref[0:CD + 1, :], tblc_ref[:, :]
        )
        residual = zf - gate_c * ci
        xaug_ref[:, 0:CD] = residual
        zfq, fb, mf = _vq_gather(
            xaug_ref[:, 0:CD + 1], fm2_ref[0:CD + 1, :], tblf_ref[:, :]
        )

        zcc = zcq + 0.1 * gate_f * fb
        zfr = zfq + gate_c * ci

        zh_ref[r0:r1, :CD] = zcc
        zh_ref[r0:r1, CD:] = zfr

        acc_ref[0:1, 0:CD] += jnp.sum(zc * zc, axis=0, keepdims=True)
        acc_ref[0:1, 0:1] += jnp.sum(mc)
        acc_ref[1:2, 0:CD] += jnp.sum(residual * residual, axis=0, keepdims=True)
        acc_ref[1:2, 0:1] += jnp.sum(mf)
        acc_ref[2:3, 0:CD] += jnp.sum(zcq, axis=0, keepdims=True)
        acc_ref[3:4, 0:CD] += jnp.sum(zcq * zcq, axis=0, keepdims=True)
        acc_ref[4:5, 0:CD] += jnp.sum(zfq, axis=0, keepdims=True)
        acc_ref[5:6, 0:CD] += jnp.sum(zfq * zfq, axis=0, keepdims=True)
        acc_ref[6:7, 0:CD] += jnp.sum(zcc, axis=0, keepdims=True) + jnp.sum(
            zfr, axis=0, keepdims=True
        )
        acc_ref[7:8, 0:CD] += jnp.sum(zcc * zcc, axis=0, keepdims=True) + jnp.sum(
            zfr * zfr, axis=0, keepdims=True
        )

    @pl.when(i == NB - 1)
    def _finish():
        sq_c = jnp.sum(acc_ref[0:1, :])
        sq_f = jnp.sum(acc_ref[1:2, :])
        s_c = jnp.sum(acc_ref[2:3, :])
        ss_c = jnp.sum(acc_ref[3:4, :])
        s_f = jnp.sum(acc_ref[4:5, :])
        ss_f = jnp.sum(acc_ref[5:6, :])
        s_h = jnp.sum(acc_ref[6:7, :])
        ss_h = jnp.sum(acc_ref[7:8, :])

        loss = 1.25 * (sq_c + sq_f) / N1
        c_info = (ss_c - s_c * s_c / N1) / (N1 - 1.0)
        f_info = (ss_f - s_f * s_f / N1) / (N1 - 1.0)
        t_info = (ss_h - s_h * s_h / N2) / (N2 - 1.0)
        compression = t_info / (c_info + f_info + 1e-8)

        ema_c = emac_ref[:, :]
        avg_c = ema_c / jnp.sum(ema_c)
        cperp = jnp.exp(-jnp.sum(avg_c * jnp.log(avg_c + 1e-10)))
        ema_f = emaf_ref[:, :]
        avg_f = ema_f / jnp.sum(ema_f)
        fperp = jnp.exp(-jnp.sum(avg_f * jnp.log(avg_f + 1e-10)))

        scal_ref[0:1, :] = jnp.broadcast_to(loss, (1, 128))
        scal_ref[1:2, :] = jnp.broadcast_to(cperp, (1, 128))
        scal_ref[2:3, :] = jnp.broadcast_to(fperp, (1, 128))
        scal_ref[3:4, :] = jnp.broadcast_to(compression, (1, 128))
        scal_ref[4:5, :] = jnp.zeros((1, 128), jnp.float32)
        scal_ref[5:6, :] = jnp.zeros((1, 128), jnp.float32)
        scal_ref[6:7, :] = jnp.zeros((1, 128), jnp.float32)
        scal_ref[7:8, :] = jnp.zeros((1, 128), jnp.float32)


def kernel(z, coarse_emb, fine_emb, c2f_W, c2f_b, c2f_gamma, c2f_beta,
           f2c_W, f2c_b, f2c_gamma, f2c_beta, coarse_gate, fine_gate,
           ema_c, ema_f):
    gates = jnp.stack([coarse_gate, fine_gate]).reshape(1, 2)

    full = lambda shape: pl.BlockSpec(shape, lambda i: (0, 0))
    zh, scal = pl.pallas_call(
        _kernel,
        grid=(NB,),
        in_specs=[
            pl.BlockSpec((BT, D), lambda i: (i, 0)),
            full((K, CD)),
            full((CD, K)),
            full((K, CD)),
            full((CD, K)),
            full((CD, CD)),
            full((1, CD)),
            full((1, CD)),
            full((1, CD)),
            full((CD, CD)),
            full((1, CD)),
            full((1, CD)),
            full((1, CD)),
            full((1, 2)),
            full((8, 128)),
            full((8, 128)),
        ],
        out_specs=[
            pl.BlockSpec((BT, D), lambda i: (i, 0)),
            full((8, 128)),
        ],
        out_shape=[
            jax.ShapeDtypeStruct((B, D), jnp.float32),
            jax.ShapeDtypeStruct((8, 128), jnp.float32),
        ],
        scratch_shapes=[
            pltpu.VMEM((8, 128), jnp.float32),
            pltpu.VMEM((K, 4 * CD), jnp.bfloat16),
            pltpu.VMEM((K, 4 * CD), jnp.bfloat16),
            pltpu.VMEM((CD + 8, K), jnp.float32),
            pltpu.VMEM((CD + 8, K), jnp.float32),
            pltpu.VMEM((BT, CD + 8), jnp.float32),
        ],
        compiler_params=pltpu.CompilerParams(
            dimension_semantics=("arbitrary",),
        ),
    )(
        z, coarse_emb, coarse_emb.T, fine_emb, fine_emb.T, c2f_W.T,
        c2f_b.reshape(1, CD), c2f_gamma.reshape(1, CD), c2f_beta.reshape(1, CD),
        f2c_W.T,
        f2c_b.reshape(1, CD), f2c_gamma.reshape(1, CD), f2c_beta.reshape(1, CD),
        gates,
        ema_c.reshape(8, 128), ema_f.reshape(8, 128),
    )

    loss = scal[0, 0]
    cperp = scal[1, 0]
    fperp = scal[2, 0]
    compression = scal[3, 0]
    return (zh, loss, cperp, fperp, compression)


# speedup vs baseline: 1.0528x; 1.0528x over previous
import jax
import jax.numpy as jnp
from jax.experimental import pallas as pl
from jax.experimental.pallas import tpu as pltpu

B = 16384
D = 128
CD = 64
K = 1024
BT = 4096
NB = B // BT
N1 = float(B * CD)
N2 = float(B * D)


def _leaky(x):
    return jnp.where(x >= 0, x, 0.1 * x)


def _split_hi_lo(x):
    hi = x.astype(jnp.bfloat16)
    lo = (x - hi.astype(jnp.float32)).astype(jnp.bfloat16)
    return hi, lo


def _make_table(emb, w_t, bias, gamma, beta):
    p = jax.lax.dot_general(
        emb, w_t, (((1,), (0,)), ((), ())), preferred_element_type=jnp.float32
    ) + bias
    m = jnp.mean(p, axis=1, keepdims=True)
    v = jnp.mean((p - m) ** 2, axis=1, keepdims=True)
    pt = (p - m) / jnp.sqrt(v + 1e-5) * gamma + beta
    e_hi, e_lo = _split_hi_lo(emb)
    pt_hi, pt_lo = _split_hi_lo(pt)
    return jnp.concatenate([e_hi, e_lo, pt_hi, pt_lo], axis=1)


def _vq_gather(x, emb_t_m2, en, tbl):
    dist = en + jax.lax.dot_general(
        x, emb_t_m2, (((1,), (0,)), ((), ())), preferred_element_type=jnp.float32
    )
    m = jnp.min(dist, axis=1, keepdims=True)
    onehot = (dist == m).astype(jnp.bfloat16)
    g = jax.lax.dot_general(
        onehot, tbl, (((1,), (0,)), ((), ())), preferred_element_type=jnp.float32
    )
    q = g[:, 0:CD] + g[:, CD:2 * CD]
    proj = _leaky(g[:, 2 * CD:3 * CD] + g[:, 3 * CD:4 * CD])
    return q, proj, m


def _kernel(
    z_ref,
    cemb_ref,
    cembt_ref,
    femb_ref,
    fembt_ref,
    c2f_w_ref,
    c2f_b_ref,
    c2f_g_ref,
    c2f_be_ref,
    f2c_w_ref,
    f2c_b_ref,
    f2c_g_ref,
    f2c_be_ref,
    gates_ref,
    emac_ref,
    emaf_ref,
    zh_ref,
    scal_ref,
    acc_ref,
    tblc_ref,
    tblf_ref,
    cm2_ref,
    fm2_ref,
):
    i = pl.program_id(0)

    @pl.when(i == 0)
    def _init():
        acc_ref[:, :] = jnp.zeros((8, 128), jnp.float32)
        tblc_ref[:, :] = _make_table(
            cemb_ref[:, :], c2f_w_ref[:, :], c2f_b_ref[0:1, :],
            c2f_g_ref[0:1, :], c2f_be_ref[0:1, :],
        )
        tblf_ref[:, :] = _make_table(
            femb_ref[:, :], f2c_w_ref[:, :], f2c_b_ref[0:1, :],
            f2c_g_ref[0:1, :], f2c_be_ref[0:1, :],
        )
        ct = cembt_ref[:, :]
        ft = fembt_ref[:, :]
        cm2_ref[0:CD, :] = -2.0 * ct
        cm2_ref[CD:CD + 1, :] = jnp.sum(ct * ct, axis=0, keepdims=True)
        cm2_ref[CD + 1:, :] = jnp.zeros((7, K), jnp.float32)
        fm2_ref[0:CD, :] = -2.0 * ft
        fm2_ref[CD:CD + 1, :] = jnp.sum(ft * ft, axis=0, keepdims=True)
        fm2_ref[CD + 1:, :] = jnp.zeros((7, K), jnp.float32)

    gate_c = jax.nn.sigmoid(gates_ref[0:1, 0:1])
    gate_f = jax.nn.sigmoid(gates_ref[0:1, 1:2])

    for h in range(1):
        r0, r1 = 0, BT
        zc = z_ref[r0:r1, :CD]
        zf = z_ref[r0:r1, CD:]

        zcq, ci, mc = _vq_gather(
            zc, cm2_ref[0:CD, :], cm2_ref[CD:CD + 1, :], tblc_ref[:, :]
        )
        residual = zf - gate_c * ci
        zfq, fb, mf = _vq_gather(
            residual, fm2_ref[0:CD, :], fm2_ref[CD:CD + 1, :], tblf_ref[:, :]
        )

        zcc = zcq + 0.1 * gate_f * fb
        zfr = zfq + gate_c * ci

        zh_ref[r0:r1, :CD] = zcc
        zh_ref[r0:r1, CD:] = zfr

        acc_ref[0:1, 0:CD] += jnp.sum(zc * zc, axis=0, keepdims=True)
        acc_ref[0:1, 0:1] += jnp.sum(mc)
        acc_ref[1:2, 0:CD] += jnp.sum(residual * residual, axis=0, keepdims=True)
        acc_ref[1:2, 0:1] += jnp.sum(mf)
        acc_ref[2:3, 0:CD] += jnp.sum(zcq, axis=0, keepdims=True)
        acc_ref[3:4, 0:CD] += jnp.sum(zcq * zcq, axis=0, keepdims=True)
        acc_ref[4:5, 0:CD] += jnp.sum(zfq, axis=0, keepdims=True)
        acc_ref[5:6, 0:CD] += jnp.sum(zfq * zfq, axis=0, keepdims=True)
        acc_ref[6:7, 0:CD] += jnp.sum(zcc, axis=0, keepdims=True) + jnp.sum(
            zfr, axis=0, keepdims=True
        )
        acc_ref[7:8, 0:CD] += jnp.sum(zcc * zcc, axis=0, keepdims=True) + jnp.sum(
            zfr * zfr, axis=0, keepdims=True
        )

    @pl.when(i == NB - 1)
    def _finish():
        sq_c = jnp.sum(acc_ref[0:1, :])
        sq_f = jnp.sum(acc_ref[1:2, :])
        s_c = jnp.sum(acc_ref[2:3, :])
        ss_c = jnp.sum(acc_ref[3:4, :])
        s_f = jnp.sum(acc_ref[4:5, :])
        ss_f = jnp.sum(acc_ref[5:6, :])
        s_h = jnp.sum(acc_ref[6:7, :])
        ss_h = jnp.sum(acc_ref[7:8, :])

        loss = 1.25 * (sq_c + sq_f) / N1
        c_info = (ss_c - s_c * s_c / N1) / (N1 - 1.0)
        f_info = (ss_f - s_f * s_f / N1) / (N1 - 1.0)
        t_info = (ss_h - s_h * s_h / N2) / (N2 - 1.0)
        compression = t_info / (c_info + f_info + 1e-8)

        ema_c = emac_ref[:, :]
        avg_c = ema_c / jnp.sum(ema_c)
        cperp = jnp.exp(-jnp.sum(avg_c * jnp.log(avg_c + 1e-10)))
        ema_f = emaf_ref[:, :]
        avg_f = ema_f / jnp.sum(ema_f)
        fperp = jnp.exp(-jnp.sum(avg_f * jnp.log(avg_f + 1e-10)))

        scal_ref[0:1, :] = jnp.broadcast_to(loss, (1, 128))
        scal_ref[1:2, :] = jnp.broadcast_to(cperp, (1, 128))
        scal_ref[2:3, :] = jnp.broadcast_to(fperp, (1, 128))
        scal_ref[3:4, :] = jnp.broadcast_to(compression, (1, 128))
        scal_ref[4:5, :] = jnp.zeros((1, 128), jnp.float32)
        scal_ref[5:6, :] = jnp.zeros((1, 128), jnp.float32)
        scal_ref[6:7, :] = jnp.zeros((1, 128), jnp.float32)
        scal_ref[7:8, :] = jnp.zeros((1, 128), jnp.float32)


def kernel(z, coarse_emb, fine_emb, c2f_W, c2f_b, c2f_gamma, c2f_beta,
           f2c_W, f2c_b, f2c_gamma, f2c_beta, coarse_gate, fine_gate,
           ema_c, ema_f):
    gates = jnp.stack([coarse_gate, fine_gate]).reshape(1, 2)

    full = lambda shape: pl.BlockSpec(shape, lambda i: (0, 0))
    zh, scal = pl.pallas_call(
        _kernel,
        grid=(NB,),
        in_specs=[
            pl.BlockSpec((BT, D), lambda i: (i, 0)),
            full((K, CD)),
            full((CD, K)),
            full((K, CD)),
            full((CD, K)),
            full((CD, CD)),
            full((1, CD)),
            full((1, CD)),
            full((1, CD)),
            full((CD, CD)),
            full((1, CD)),
            full((1, CD)),
            full((1, CD)),
            full((1, 2)),
            full((8, 128)),
            full((8, 128)),
        ],
        out_specs=[
            pl.BlockSpec((BT, D), lambda i: (i, 0)),
            full((8, 128)),
        ],
        out_shape=[
            jax.ShapeDtypeStruct((B, D), jnp.float32),
            jax.ShapeDtypeStruct((8, 128), jnp.float32),
        ],
        scratch_shapes=[
            pltpu.VMEM((8, 128), jnp.float32),
            pltpu.VMEM((K, 4 * CD), jnp.bfloat16),
            pltpu.VMEM((K, 4 * CD), jnp.bfloat16),
            pltpu.VMEM((CD + 8, K), jnp.float32),
            pltpu.VMEM((CD + 8, K), jnp.float32),
        ],
        compiler_params=pltpu.CompilerParams(
            dimension_semantics=("arbitrary",),
        ),
    )(
        z, coarse_emb, coarse_emb.T, fine_emb, fine_emb.T, c2f_W.T,
        c2f_b.reshape(1, CD), c2f_gamma.reshape(1, CD), c2f_beta.reshape(1, CD),
        f2c_W.T,
        f2c_b.reshape(1, CD), f2c_gamma.reshape(1, CD), f2c_beta.reshape(1, CD),
        gates,
        ema_c.reshape(8, 128), ema_f.reshape(8, 128),
    )

    loss = scal[0, 0]
    cperp = scal[1, 0]
    fperp = scal[2, 0]
    compression = scal[3, 0]
    return (zh, loss, cperp, fperp, compression)


# speedup vs baseline: 1.1294x; 1.0727x over previous
import jax
import jax.numpy as jnp
from jax.experimental import pallas as pl
from jax.experimental.pallas import tpu as pltpu

B = 16384
D = 128
CD = 64
K = 1024
BT = 4096
NB = B // BT
N1 = float(B * CD)
N2 = float(B * D)


def _leaky(x):
    return jnp.where(x >= 0, x, 0.1 * x)


def _proj_table(emb, w_t, bias, gamma, beta, scale):
    p = jax.lax.dot_general(
        emb, w_t, (((1,), (0,)), ((), ())), preferred_element_type=jnp.float32
    ) + bias
    m = jnp.mean(p, axis=1, keepdims=True)
    v = jnp.mean((p - m) ** 2, axis=1, keepdims=True)
    return scale * _leaky((p - m) / jnp.sqrt(v + 1e-5) * gamma + beta)


def _rs(x):
    return jnp.sum(x, axis=1, keepdims=True)


def _vq_gather(x, emb_t_m2, en, tbl, n_out):
    dist = en + jax.lax.dot_general(
        x, emb_t_m2, (((1,), (0,)), ((), ())), preferred_element_type=jnp.float32
    )
    m = jnp.min(dist, axis=1, keepdims=True)
    onehot = (dist == m).astype(jnp.bfloat16)
    g = jax.lax.dot_general(
        onehot, tbl, (((1,), (0,)), ((), ())), preferred_element_type=jnp.float32
    )
    return g, m


def _kernel(
    z_ref,
    cemb_ref,
    cembt_ref,
    femb_ref,
    fembt_ref,
    c2f_w_ref,
    c2f_b_ref,
    c2f_g_ref,
    c2f_be_ref,
    f2c_w_ref,
    f2c_b_ref,
    f2c_g_ref,
    f2c_be_ref,
    gates_ref,
    emac_ref,
    emaf_ref,
    zh_ref,
    scal_ref,
    acc_ref,
    tblc_ref,
    tblf_ref,
    cm2_ref,
    fm2_ref,
):
    i = pl.program_id(0)

    gate_c = jax.nn.sigmoid(gates_ref[0:1, 0:1])
    gate_f = jax.nn.sigmoid(gates_ref[0:1, 1:2])

    @pl.when(i == 0)
    def _init():
        acc_ref[:, :] = jnp.zeros((8, 128), jnp.float32)
        ec = cemb_ref[:, :]
        g = gate_c * _proj_table(
            ec, c2f_w_ref[:, :], c2f_b_ref[0:1, :],
            c2f_g_ref[0:1, :], c2f_be_ref[0:1, :], 1.0,
        )
        zpad = jnp.zeros((K, CD - 4), jnp.float32)
        tblc_ref[:, :] = jnp.concatenate(
            [
                ec.astype(jnp.bfloat16).astype(jnp.float32),
                g.astype(jnp.bfloat16).astype(jnp.float32),
                g - g.astype(jnp.bfloat16).astype(jnp.float32),
                _rs(ec), _rs(ec * ec), _rs(g), _rs(g * g),
                zpad,
            ],
            axis=1,
        ).astype(jnp.bfloat16)
        ef = femb_ref[:, :]
        h = gate_f * _proj_table(
            ef, f2c_w_ref[:, :], f2c_b_ref[0:1, :],
            f2c_g_ref[0:1, :], f2c_be_ref[0:1, :], 0.1,
        )
        tblf_ref[:, :] = jnp.concatenate([ef, h], axis=1).astype(jnp.bfloat16)

        ct = cembt_ref[:, :]
        ft = fembt_ref[:, :]
        cm2_ref[0:CD, :] = -2.0 * ct
        cm2_ref[CD:CD + 1, :] = jnp.sum(ct * ct, axis=0, keepdims=True)
        cm2_ref[CD + 1:, :] = jnp.zeros((7, K), jnp.float32)
        fm2_ref[0:CD, :] = -2.0 * ft
        fm2_ref[CD:CD + 1, :] = jnp.sum(ft * ft, axis=0, keepdims=True)
        fm2_ref[CD + 1:, :] = jnp.zeros((7, K), jnp.float32)

    zc = z_ref[:, :CD]
    zf = z_ref[:, CD:]

    gc_out, mc = _vq_gather(
        zc, cm2_ref[0:CD, :], cm2_ref[CD:CD + 1, :], tblc_ref[:, :], 4 * CD
    )
    zcq = gc_out[:, 0:CD]
    g = gc_out[:, CD:2 * CD] + gc_out[:, 2 * CD:3 * CD]
    residual = zf - g
    gf_out, mf = _vq_gather(
        residual, fm2_ref[0:CD, :], fm2_ref[CD:CD + 1, :], tblf_ref[:, :], 2 * CD
    )
    zfq = gf_out[:, 0:CD]
    h = gf_out[:, CD:2 * CD]

    zcc = zcq + h
    zfr = zfq + g

    zh_ref[:, :CD] = zcc
    zh_ref[:, CD:] = zfr

    acc_ref[0:1, 0:CD] += jnp.sum(zc * zc + residual * residual, axis=0,
                                  keepdims=True)
    acc_ref[0:1, 0:1] += jnp.sum(mc) + jnp.sum(mf)
    acc_ref[1:2, 0:4] += jnp.sum(gc_out[:, 3 * CD:3 * CD + 4], axis=0,
                                 keepdims=True)
    acc_ref[2:3, 0:CD] += jnp.sum(zfq, axis=0, keepdims=True)
    acc_ref[3:4, 0:CD] += jnp.sum(zfq * zfq, axis=0, keepdims=True)
    acc_ref[4:5, 0:CD] += jnp.sum(h, axis=0, keepdims=True)
    acc_ref[5:6, 0:CD] += jnp.sum(zcq * h + zfq * g, axis=0, keepdims=True)
    acc_ref[6:7, 0:CD] += jnp.sum(h * h, axis=0, keepdims=True)

    @pl.when(i == NB - 1)
    def _finish():
        sq = jnp.sum(acc_ref[0:1, :])
        s_c = acc_ref[1, 0]
        ss_c = acc_ref[1, 1]
        s_g = acc_ref[1, 2]
        ss_g = acc_ref[1, 3]
        s_f = jnp.sum(acc_ref[2:3, :])
        ss_f = jnp.sum(acc_ref[3:4, :])
        s_hh = jnp.sum(acc_ref[4:5, :])
        cross = jnp.sum(acc_ref[5:6, :])
        ss_hh = jnp.sum(acc_ref[6:7, :])

        loss = 1.25 * sq / N1
        c_info = (ss_c - s_c * s_c / N1) / (N1 - 1.0)
        f_info = (ss_f - s_f * s_f / N1) / (N1 - 1.0)
        s_h = s_c + s_f + s_g + s_hh
        ss_h = ss_c + ss_f + ss_g + ss_hh + 2.0 * cross
        t_info = (ss_h - s_h * s_h / N2) / (N2 - 1.0)
        compression = t_info / (c_info + f_info + 1e-8)

        ema_c = emac_ref[:, :]
        avg_c = ema_c / jnp.sum(ema_c)
        cperp = jnp.exp(-jnp.sum(avg_c * jnp.log(avg_c + 1e-10)))
        ema_f = emaf_ref[:, :]
        avg_f = ema_f / jnp.sum(ema_f)
        fperp = jnp.exp(-jnp.sum(avg_f * jnp.log(avg_f + 1e-10)))

        scal_ref[0:1, :] = jnp.broadcast_to(loss, (1, 128))
        scal_ref[1:2, :] = jnp.broadcast_to(cperp, (1, 128))
        scal_ref[2:3, :] = jnp.broadcast_to(fperp, (1, 128))
        scal_ref[3:4, :] = jnp.broadcast_to(compression, (1, 128))
        scal_ref[4:5, :] = jnp.zeros((1, 128), jnp.float32)
        scal_ref[5:6, :] = jnp.zeros((1, 128), jnp.float32)
        scal_ref[6:7, :] = jnp.zeros((1, 128), jnp.float32)
        scal_ref[7:8, :] = jnp.zeros((1, 128), jnp.float32)


def kernel(z, coarse_emb, fine_emb, c2f_W, c2f_b, c2f_gamma, c2f_beta,
           f2c_W, f2c_b, f2c_gamma, f2c_beta, coarse_gate, fine_gate,
           ema_c, ema_f):
    gates = jnp.stack([coarse_gate, fine_gate]).reshape(1, 2)

    full = lambda shape: pl.BlockSpec(shape, lambda i: (0, 0))
    zh, scal = pl.pallas_call(
        _kernel,
        grid=(NB,),
        in_specs=[
            pl.BlockSpec((BT, D), lambda i: (i, 0)),
            full((K, CD)),
            full((CD, K)),
            full((K, CD)),
            full((CD, K)),
            full((CD, CD)),
            full((1, CD)),
            full((1, CD)),
            full((1, CD)),
            full((CD, CD)),
            full((1, CD)),
            full((1, CD)),
            full((1, CD)),
            full((1, 2)),
            full((8, 128)),
            full((8, 128)),
        ],
        out_specs=[
            pl.BlockSpec((BT, D), lambda i: (i, 0)),
            full((8, 128)),
        ],
        out_shape=[
            jax.ShapeDtypeStruct((B, D), jnp.float32),
            jax.ShapeDtypeStruct((8, 128), jnp.float32),
        ],
        scratch_shapes=[
            pltpu.VMEM((8, 128), jnp.float32),
            pltpu.VMEM((K, 4 * CD), jnp.bfloat16),
            pltpu.VMEM((K, 2 * CD), jnp.bfloat16),
            pltpu.VMEM((CD + 8, K), jnp.float32),
            pltpu.VMEM((CD + 8, K), jnp.float32),
        ],
        compiler_params=pltpu.CompilerParams(
            dimension_semantics=("arbitrary",),
        ),
    )(
        z, coarse_emb, coarse_emb.T, fine_emb, fine_emb.T, c2f_W.T,
        c2f_b.reshape(1, CD), c2f_gamma.reshape(1, CD), c2f_beta.reshape(1, CD),
        f2c_W.T,
        f2c_b.reshape(1, CD), f2c_gamma.reshape(1, CD), f2c_beta.reshape(1, CD),
        gates,
        ema_c.reshape(8, 128), ema_f.reshape(8, 128),
    )

    loss = scal[0, 0]
    cperp = scal[1, 0]
    fperp = scal[2, 0]
    compression = scal[3, 0]
    return (zh, loss, cperp, fperp, compression)
